# baseline (speedup 1.0000x reference)
import functools

import jax
import jax.numpy as jnp
import numpy as np
from jax import lax
from jax.experimental import pallas as pl
from jax.experimental.pallas import tpu as pltpu
from jax.experimental.pallas import tpu_sc as plsc

N_NODES = 10000
N_EDGES = 320000
NFEAT = 128
NHID = 128
NCLASS = 40
NCLS_PAD = 64

CHUNK = 128
NCHUNK2 = 162
E_PAD = 16 * NCHUNK2 * CHUNK
DRAIN = 80
NDRAIN = N_NODES // DRAIN


def _unpack_perm(D):
  t = np.empty(D, np.int32)
  for g in range(D // 32):
    for i in range(16):
      t[32 * g + i] = 16 * g + i
      t[32 * g + 16 + i] = D // 2 + 16 * g + i
  return t


_INV_T128 = np.argsort(_unpack_perm(NHID))
_INV_T64 = np.argsort(_unpack_perm(NCLS_PAD))


def _make_spmm(D, n0):
  n1 = NCHUNK2 - n0
  assert n0 % 2 == 0 and n1 % 2 == 0
  mesh = plsc.VectorSubcoreMesh(
      core_axis_name="c", subcore_axis_name="s", num_cores=2, num_subcores=16)

  @functools.partial(
      pl.kernel,
      out_type=jax.ShapeDtypeStruct((2, N_NODES, D), jnp.float32),
      mesh=mesh,
      scratch_types=[
          [pltpu.VMEM((1, CHUNK), jnp.int32) for _ in range(2)],
          [pltpu.VMEM((1, CHUNK), jnp.int32) for _ in range(2)],
          [pltpu.VMEM((CHUNK,), jnp.float32) for _ in range(2)],
          [pltpu.VMEM((CHUNK, D // 2), jnp.int32) for _ in range(2)],
          pltpu.VMEM((CHUNK, D), jnp.float32),
          [pltpu.SemaphoreType.DMA for _ in range(8)],
          pltpu.VMEM_SHARED((N_NODES, D), jnp.float32),
      ],
      compiler_params=pltpu.CompilerParams(use_tc_tiling_on_sc=False,
                                           needs_layout_passes=False),
  )
  def spmm(sup_hbm, src_hbm, dst_hbm, ew_hbm, out_hbm,
           srcv, dstv, ewv, rows, frows, sems, acc):
    c = lax.axis_index("c")
    s = lax.axis_index("s")
    ssr = sems[0:2]
    sds = sems[2:4]
    sew = sems[4:6]
    sg = sems[6:8]
    n_k = jnp.where(c == 0, n0, n1)
    n_rr = jnp.where(s < NDRAIN % 16, NDRAIN // 16 + 1, NDRAIN // 16)

    def zero_rows(i, _):
      for g in range(D // 16):
        frows[i, pl.ds(16 * g, 16)] = jnp.zeros((16,), jnp.float32)
      return 0
    lax.fori_loop(0, DRAIN, zero_rows, 0)

    def zero_acc(j, _):
      pltpu.sync_copy(frows.at[pl.ds(0, DRAIN)],
                      acc.at[pl.ds((s + j * 16) * DRAIN, DRAIN)])
      return 0
    lax.fori_loop(0, n_rr, zero_acc, 0)
    plsc.subcore_barrier()

    def issue_idx(k, b):
      pltpu.async_copy(src_hbm.at[c].at[s].at[k], srcv[b], ssr[b])
      pltpu.async_copy(dst_hbm.at[c].at[s].at[k], dstv[b], sds[b])
      pltpu.async_copy(ew_hbm.at[c].at[s].at[pl.ds(k * CHUNK, CHUNK)],
                       ewv[b], sew[b])

    def wait_idx(k, b):
      pltpu.make_async_copy(src_hbm.at[c].at[s].at[k], srcv[b], ssr[b]).wait()
      pltpu.make_async_copy(dst_hbm.at[c].at[s].at[k], dstv[b], sds[b]).wait()
      pltpu.make_async_copy(ew_hbm.at[c].at[s].at[pl.ds(k * CHUNK, CHUNK)],
                            ewv[b], sew[b]).wait()

    def issue_gather(b):
      pltpu.async_copy(sup_hbm.at[srcv[b].at[0]], rows[b], sg[b])

    def wait_gather(b):
      pltpu.make_async_copy(sup_hbm.at[srcv[b].at[0]], rows[b], sg[b]).wait()

    def scale_scatter(b):
      @plsc.parallel_loop(0, CHUNK, unroll=4)
      def edge_body(i):
        w16 = plsc.load_gather(ewv[b], [jnp.full((16,), i, jnp.int32)])
        for g in range(D // 32):
          ab = rows[b][i, pl.ds(16 * g, 16)]
          lo = plsc.bitcast(jnp.left_shift(ab, 16), jnp.float32)
          hi = plsc.bitcast(jnp.bitwise_and(ab, jnp.int32(-65536)),
                            jnp.float32)
          frows[i, pl.ds(32 * g, 16)] = lo * w16
          frows[i, pl.ds(32 * g + 16, 16)] = hi * w16
      pltpu.sync_copy(frows, acc.at[dstv[b].at[0]], add=True)

    issue_idx(0, 0)
    wait_idx(0, 0)
    issue_idx(1, 1)
    issue_gather(0)

    def pipe_body(j, _):
      for b in range(2):
        k = 2 * j + b
        nb = 1 - b
        wait_idx(k + 1, nb)
        issue_gather(nb)
        wait_gather(b)
        scale_scatter(b)
        issue_idx(k + 2, b)
      return 0
    lax.fori_loop(0, n_k // 2 - 1, pipe_body, 0)

    wait_idx(n_k - 1, 1)
    issue_gather(1)
    wait_gather(0)
    scale_scatter(0)
    wait_gather(1)
    scale_scatter(1)

    plsc.subcore_barrier()

    def drain(j, _):
      base = (s + j * 16) * DRAIN
      pltpu.sync_copy(acc.at[pl.ds(base, DRAIN)], frows.at[pl.ds(0, DRAIN)])
      pltpu.sync_copy(frows.at[pl.ds(0, DRAIN)],
                      out_hbm.at[c].at[pl.ds(base, DRAIN)])
      return 0
    lax.fori_loop(0, n_rr, drain, 0)

  return spmm


_N0_L1 = 68
_N0_L2 = 76

_spmm128 = _make_spmm(NHID, _N0_L1)
_spmm64 = _make_spmm(NCLS_PAD, _N0_L2)

_RB = 1000


def _pack_bf16(r):
  h = r.shape[1] // 2
  v = lax.bitcast_convert_type(r, jnp.uint32) + jnp.uint32(0x8000)
  packed = (v[:, :h] >> 16) | (v[:, h:] & jnp.uint32(0xFFFF0000))
  return lax.bitcast_convert_type(packed, jnp.int32)


def _mm1_body(x_ref, w_ref, o_ref):
  o_ref[...] = _pack_bf16(jnp.dot(x_ref[...], w_ref[...],
                                  preferred_element_type=jnp.float32))


def _mm1(x, W1):
  return pl.pallas_call(
      _mm1_body,
      grid=(N_NODES // _RB,),
      in_specs=[
          pl.BlockSpec((_RB, NFEAT), lambda i: (i, 0)),
          pl.BlockSpec((NFEAT, NHID), lambda i: (0, 0)),
      ],
      out_specs=pl.BlockSpec((_RB, NHID // 2), lambda i: (i, 0)),
      out_shape=jax.ShapeDtypeStruct((N_NODES, NHID // 2), jnp.int32),
  )(x, W1)


def _combine_body(p0_ref, p1_ref, b_ref, w_ref, o_ref):
  h = jnp.maximum(p0_ref[...] + p1_ref[...] + b_ref[...], 0.0)
  o_ref[...] = _pack_bf16(jnp.dot(h, w_ref[...],
                                  preferred_element_type=jnp.float32))


def _combine(p0, p1, b1, W2p):
  return pl.pallas_call(
      _combine_body,
      grid=(N_NODES // _RB,),
      in_specs=[
          pl.BlockSpec((_RB, NHID), lambda i: (i, 0)),
          pl.BlockSpec((_RB, NHID), lambda i: (i, 0)),
          pl.BlockSpec((1, NHID), lambda i: (0, 0)),
          pl.BlockSpec((NHID, NCLS_PAD), lambda i: (0, 0)),
      ],
      out_specs=pl.BlockSpec((_RB, NCLS_PAD // 2), lambda i: (i, 0)),
      out_shape=jax.ShapeDtypeStruct((N_NODES, NCLS_PAD // 2), jnp.int32),
  )(p0, p1, b1.reshape(1, NHID), W2p)


def _final_body(p0_ref, p1_ref, b_ref, o_ref):
  z = (p0_ref[...] + p1_ref[...])[:, :NCLASS] + b_ref[...]
  z = z - jnp.max(z, axis=1, keepdims=True)
  e = jnp.exp(z)
  o_ref[...] = e / jnp.sum(e, axis=1, keepdims=True)


def _final(p0, p1, b2):
  return pl.pallas_call(
      _final_body,
      grid=(N_NODES // _RB,),
      in_specs=[
          pl.BlockSpec((_RB, NCLS_PAD), lambda i: (i, 0)),
          pl.BlockSpec((_RB, NCLS_PAD), lambda i: (i, 0)),
          pl.BlockSpec((1, NCLASS), lambda i: (0, 0)),
      ],
      out_specs=pl.BlockSpec((_RB, NCLASS), lambda i: (i, 0)),
      out_shape=jax.ShapeDtypeStruct((N_NODES, NCLASS), jnp.float32),
  )(p0, p1, b2.reshape(1, NCLASS))


def _split_edges(flat, n0):
  n1 = NCHUNK2 - n0
  nm = max(n0, n1)
  a0 = flat[:16 * n0 * CHUNK].reshape(16, n0 * CHUNK)
  a0 = jnp.pad(a0, ((0, 0), (0, (nm - n0) * CHUNK)))
  a1 = flat[16 * n0 * CHUNK:].reshape(16, n1 * CHUNK)
  a1 = jnp.pad(a1, ((0, 0), (0, (nm - n1) * CHUNK)))
  return jnp.stack([a0, a1])


@jax.jit
def kernel(x, edge_index, edge_weight, W1, b1, W2, b2):
  ei = jnp.pad(edge_index.astype(jnp.int32), ((0, 0), (0, E_PAD - N_EDGES)))
  ewf = jnp.pad(edge_weight, (0, E_PAD - N_EDGES))

  def prep(n0):
    nm = max(n0, NCHUNK2 - n0)
    src = _split_edges(ei[0], n0).reshape(2, 16, nm, 1, CHUNK)
    dst = _split_edges(ei[1], n0).reshape(2, 16, nm, 1, CHUNK)
    ew = _split_edges(ewf, n0)
    return src, dst, ew

  src1, dst1, ew1 = prep(_N0_L1)
  src2, dst2, ew2 = prep(_N0_L2)

  W1p = W1[:, _INV_T128]
  W2p = jnp.pad(W2, ((0, 0), (0, NCLS_PAD - NCLASS)))[:, _INV_T64]

  sup1 = _mm1(x, W1p)
  p1 = _spmm128(sup1, src1, dst1, ew1)
  sup2 = _combine(p1[0], p1[1], b1, W2p)
  p2 = _spmm64(sup2, src2, dst2, ew2)
  return _final(p2[0], p2[1], b2)


# speedup vs baseline: 1.2237x; 1.2237x over previous
import functools

import jax
import jax.numpy as jnp
import numpy as np
from jax import lax
from jax.experimental import pallas as pl
from jax.experimental.pallas import tpu as pltpu
from jax.experimental.pallas import tpu_sc as plsc

N_NODES = 10000
N_EDGES = 320000
NFEAT = 128
NHID = 128
NCLASS = 40
NCLS_PAD = 64

CHUNK = 128
NCHUNK2 = 160
E_PAD = 16 * NCHUNK2 * CHUNK
DRAIN = 80
NDRAIN = N_NODES // DRAIN


def _unpack_perm(D):
  t = np.empty(D, np.int32)
  for g in range(D // 32):
    for i in range(16):
      t[32 * g + i] = 16 * g + i
      t[32 * g + 16 + i] = D // 2 + 16 * g + i
  return t


_INV_T128 = np.argsort(_unpack_perm(NHID))
_INV_T64 = np.argsort(_unpack_perm(NCLS_PAD))


def _make_spmm(D, n0):
  n1 = NCHUNK2 - n0
  assert n0 % 2 == 0 and n1 % 2 == 0
  mesh = plsc.VectorSubcoreMesh(
      core_axis_name="c", subcore_axis_name="s", num_cores=2, num_subcores=16)

  @functools.partial(
      pl.kernel,
      out_type=jax.ShapeDtypeStruct((2, N_NODES, D), jnp.float32),
      mesh=mesh,
      scratch_types=[
          [pltpu.VMEM((1, CHUNK), jnp.int32) for _ in range(2)],
          [pltpu.VMEM((1, CHUNK), jnp.int32) for _ in range(2)],
          [pltpu.VMEM((CHUNK,), jnp.float32) for _ in range(2)],
          [pltpu.VMEM((CHUNK, D // 2), jnp.int32) for _ in range(2)],
          pltpu.VMEM((CHUNK, D), jnp.float32),
          [pltpu.SemaphoreType.DMA for _ in range(8)],
          pltpu.VMEM_SHARED((N_NODES, D), jnp.float32),
      ],
      compiler_params=pltpu.CompilerParams(use_tc_tiling_on_sc=False,
                                           needs_layout_passes=False),
  )
  def spmm(sup_hbm, src_hbm, dst_hbm, ew_hbm, out_hbm,
           srcv, dstv, ewv, rows, frows, sems, acc):
    c = lax.axis_index("c")
    s = lax.axis_index("s")
    ssr = sems[0:2]
    sds = sems[2:4]
    sew = sems[4:6]
    sg = sems[6:8]
    n_k = jnp.where(c == 0, n0, n1)
    n_rr = jnp.where(s < NDRAIN % 16, NDRAIN // 16 + 1, NDRAIN // 16)

    def zero_rows(i, _):
      for g in range(D // 16):
        frows[i, pl.ds(16 * g, 16)] = jnp.zeros((16,), jnp.float32)
      return 0
    lax.fori_loop(0, DRAIN, zero_rows, 0)

    def zero_acc(j, _):
      pltpu.sync_copy(frows.at[pl.ds(0, DRAIN)],
                      acc.at[pl.ds((s + j * 16) * DRAIN, DRAIN)])
      return 0
    lax.fori_loop(0, n_rr, zero_acc, 0)
    plsc.subcore_barrier()

    def issue_idx(k, b):
      pltpu.async_copy(src_hbm.at[c].at[s].at[k], srcv[b], ssr[b])
      pltpu.async_copy(dst_hbm.at[c].at[s].at[k], dstv[b], sds[b])
      pltpu.async_copy(ew_hbm.at[c].at[s].at[pl.ds(k * CHUNK, CHUNK)],
                       ewv[b], sew[b])

    def wait_idx(k, b):
      pltpu.make_async_copy(src_hbm.at[c].at[s].at[k], srcv[b], ssr[b]).wait()
      pltpu.make_async_copy(dst_hbm.at[c].at[s].at[k], dstv[b], sds[b]).wait()
      pltpu.make_async_copy(ew_hbm.at[c].at[s].at[pl.ds(k * CHUNK, CHUNK)],
                            ewv[b], sew[b]).wait()

    def issue_gather(b):
      pltpu.async_copy(sup_hbm.at[srcv[b].at[0]], rows[b], sg[b])

    def wait_gather(b):
      pltpu.make_async_copy(sup_hbm.at[srcv[b].at[0]], rows[b], sg[b]).wait()

    def scale_scatter(b):
      @plsc.parallel_loop(0, CHUNK, unroll=4)
      def edge_body(i):
        w16 = plsc.load_gather(ewv[b], [jnp.full((16,), i, jnp.int32)])
        for g in range(D // 32):
          ab = rows[b][i, pl.ds(16 * g, 16)]
          lo = plsc.bitcast(jnp.left_shift(ab, 16), jnp.float32)
          hi = plsc.bitcast(jnp.bitwise_and(ab, jnp.int32(-65536)),
                            jnp.float32)
          frows[i, pl.ds(32 * g, 16)] = lo * w16
          frows[i, pl.ds(32 * g + 16, 16)] = hi * w16
      pltpu.sync_copy(frows, acc.at[dstv[b].at[0]], add=True)

    issue_idx(0, 0)
    wait_idx(0, 0)
    issue_idx(1, 1)
    issue_gather(0)

    def pipe_body(j, _):
      for b in range(2):
        k = 2 * j + b
        nb = 1 - b
        wait_idx(k + 1, nb)
        issue_gather(nb)
        wait_gather(b)
        scale_scatter(b)
        issue_idx(k + 2, b)
      return 0
    lax.fori_loop(0, n_k // 2 - 1, pipe_body, 0)

    wait_idx(n_k - 1, 1)
    issue_gather(1)
    wait_gather(0)
    scale_scatter(0)
    wait_gather(1)
    scale_scatter(1)

    plsc.subcore_barrier()

    def drain(j, _):
      base = (s + j * 16) * DRAIN
      pltpu.sync_copy(acc.at[pl.ds(base, DRAIN)], frows.at[pl.ds(0, DRAIN)])
      pltpu.sync_copy(frows.at[pl.ds(0, DRAIN)],
                      out_hbm.at[c].at[pl.ds(base, DRAIN)])
      return 0
    lax.fori_loop(0, n_rr, drain, 0)

  return spmm


_N0_L1 = 80
_N0_L2 = 80

_spmm128 = _make_spmm(NHID, _N0_L1)
_spmm64 = _make_spmm(NCLS_PAD, _N0_L2)

_RB = 1000


def _pack_bf16(r):
  h = r.shape[1] // 2
  v = lax.bitcast_convert_type(r, jnp.uint32) + jnp.uint32(0x8000)
  packed = (v[:, :h] >> 16) | (v[:, h:] & jnp.uint32(0xFFFF0000))
  return lax.bitcast_convert_type(packed, jnp.int32)


def _mm1_body(x_ref, w_ref, o_ref):
  o_ref[...] = _pack_bf16(jnp.dot(x_ref[...], w_ref[...],
                                  preferred_element_type=jnp.float32))


def _mm1(x, W1):
  return pl.pallas_call(
      _mm1_body,
      grid=(N_NODES // _RB,),
      in_specs=[
          pl.BlockSpec((_RB, NFEAT), lambda i: (i, 0)),
          pl.BlockSpec((NFEAT, NHID), lambda i: (0, 0)),
      ],
      out_specs=pl.BlockSpec((_RB, NHID // 2), lambda i: (i, 0)),
      out_shape=jax.ShapeDtypeStruct((N_NODES, NHID // 2), jnp.int32),
  )(x, W1)


def _combine_body(p0_ref, p1_ref, b_ref, w_ref, o_ref):
  h = jnp.maximum(p0_ref[...] + p1_ref[...] + b_ref[...], 0.0)
  o_ref[...] = _pack_bf16(jnp.dot(h, w_ref[...],
                                  preferred_element_type=jnp.float32))


def _combine(p0, p1, b1, W2p):
  return pl.pallas_call(
      _combine_body,
      grid=(N_NODES // _RB,),
      in_specs=[
          pl.BlockSpec((_RB, NHID), lambda i: (i, 0)),
          pl.BlockSpec((_RB, NHID), lambda i: (i, 0)),
          pl.BlockSpec((1, NHID), lambda i: (0, 0)),
          pl.BlockSpec((NHID, NCLS_PAD), lambda i: (0, 0)),
      ],
      out_specs=pl.BlockSpec((_RB, NCLS_PAD // 2), lambda i: (i, 0)),
      out_shape=jax.ShapeDtypeStruct((N_NODES, NCLS_PAD // 2), jnp.int32),
  )(p0, p1, b1.reshape(1, NHID), W2p)


def _final_body(p0_ref, p1_ref, b_ref, o_ref):
  z = (p0_ref[...] + p1_ref[...])[:, :NCLASS] + b_ref[...]
  z = z - jnp.max(z, axis=1, keepdims=True)
  e = jnp.exp(z)
  o_ref[...] = e / jnp.sum(e, axis=1, keepdims=True)


def _final(p0, p1, b2):
  return pl.pallas_call(
      _final_body,
      grid=(N_NODES // _RB,),
      in_specs=[
          pl.BlockSpec((_RB, NCLS_PAD), lambda i: (i, 0)),
          pl.BlockSpec((_RB, NCLS_PAD), lambda i: (i, 0)),
          pl.BlockSpec((1, NCLASS), lambda i: (0, 0)),
      ],
      out_specs=pl.BlockSpec((_RB, NCLASS), lambda i: (i, 0)),
      out_shape=jax.ShapeDtypeStruct((N_NODES, NCLASS), jnp.float32),
  )(p0, p1, b2.reshape(1, NCLASS))


def _split_edges(flat, n0):
  n1 = NCHUNK2 - n0
  nm = max(n0, n1)
  a0 = flat[:16 * n0 * CHUNK].reshape(16, n0 * CHUNK)
  a0 = jnp.pad(a0, ((0, 0), (0, (nm - n0) * CHUNK)))
  a1 = flat[16 * n0 * CHUNK:].reshape(16, n1 * CHUNK)
  a1 = jnp.pad(a1, ((0, 0), (0, (nm - n1) * CHUNK)))
  return jnp.stack([a0, a1])


@jax.jit
def kernel(x, edge_index, edge_weight, W1, b1, W2, b2):
  ei = jnp.pad(edge_index.astype(jnp.int32), ((0, 0), (0, E_PAD - N_EDGES)))
  ewf = jnp.pad(edge_weight, (0, E_PAD - N_EDGES))

  def prep(n0):
    nm = max(n0, NCHUNK2 - n0)
    src = _split_edges(ei[0], n0).reshape(2, 16, nm, 1, CHUNK)
    dst = _split_edges(ei[1], n0).reshape(2, 16, nm, 1, CHUNK)
    ew = _split_edges(ewf, n0)
    return src, dst, ew

  src1, dst1, ew1 = prep(_N0_L1)
  src2, dst2, ew2 = prep(_N0_L2)

  W1p = W1[:, _INV_T128]
  W2p = jnp.pad(W2, ((0, 0), (0, NCLS_PAD - NCLASS)))[:, _INV_T64]

  sup1 = _mm1(x, W1p)
  p1 = _spmm128(sup1, src1, dst1, ew1)
  sup2 = _combine(p1[0], p1[1], b1, W2p)
  p2 = _spmm64(sup2, src2, dst2, ew2)
  return _final(p2[0], p2[1], b2)


# speedup vs baseline: 1.3468x; 1.1006x over previous
import functools

import jax
import jax.numpy as jnp
import numpy as np
from jax import lax
from jax.experimental import pallas as pl
from jax.experimental.pallas import tpu as pltpu
from jax.experimental.pallas import tpu_sc as plsc

N_NODES = 10000
N_EDGES = 320000
NFEAT = 128
NHID = 128
NCLASS = 40
NCLS_PAD = 64

CHUNK = 128
NCHUNK = 80
NW = 32
EDGES_PER_W = NCHUNK * CHUNK
E_PAD = NW * EDGES_PER_W
DRAIN = 80
NDRAIN = N_NODES // DRAIN


def _unpack_perm(D):
  t = np.empty(D, np.int32)
  for g in range(D // 32):
    for i in range(16):
      t[32 * g + i] = 16 * g + i
      t[32 * g + 16 + i] = D // 2 + 16 * g + i
  return t


_INV_T128 = np.argsort(_unpack_perm(NHID))
_INV_T64 = np.argsort(_unpack_perm(NCLS_PAD))


def _make_spmm(D):
  mesh = plsc.VectorSubcoreMesh(
      core_axis_name="c", subcore_axis_name="s", num_cores=2, num_subcores=16)

  @functools.partial(
      pl.kernel,
      out_type=jax.ShapeDtypeStruct((2, N_NODES, D), jnp.float32),
      mesh=mesh,
      scratch_types=[
          [pltpu.VMEM((1, CHUNK), jnp.int32) for _ in range(2)],
          [pltpu.VMEM((1, CHUNK), jnp.int32) for _ in range(2)],
          [pltpu.VMEM((CHUNK,), jnp.float32) for _ in range(2)],
          [pltpu.VMEM((CHUNK, D // 2), jnp.int32) for _ in range(2)],
          pltpu.VMEM((CHUNK, D), jnp.float32),
          [pltpu.SemaphoreType.DMA for _ in range(8)],
          pltpu.VMEM_SHARED((N_NODES, D), jnp.float32),
      ],
      compiler_params=pltpu.CompilerParams(use_tc_tiling_on_sc=False,
                                           needs_layout_passes=False),
  )
  def spmm(sup_hbm, src_hbm, dst_hbm, ew_hbm, out_hbm,
           srcv, dstv, ewv, rows, frows, sems, acc):
    c = lax.axis_index("c")
    s = lax.axis_index("s")
    w = s * 2 + c
    ssr = sems[0:2]
    sds = sems[2:4]
    sew = sems[4:6]
    sg = sems[6:8]
    n_rr = jnp.where(s < NDRAIN % 16, NDRAIN // 16 + 1, NDRAIN // 16)

    def zero_rows(i, _):
      for g in range(D // 16):
        frows[i, pl.ds(16 * g, 16)] = jnp.zeros((16,), jnp.float32)
      return 0
    lax.fori_loop(0, DRAIN, zero_rows, 0)

    def zero_acc(j, _):
      pltpu.sync_copy(frows.at[pl.ds(0, DRAIN)],
                      acc.at[pl.ds((s + j * 16) * DRAIN, DRAIN)])
      return 0
    lax.fori_loop(0, n_rr, zero_acc, 0)
    plsc.subcore_barrier()

    def issue_idx(k, b):
      pltpu.async_copy(src_hbm.at[w].at[k], srcv[b], ssr[b])
      pltpu.async_copy(dst_hbm.at[w].at[k], dstv[b], sds[b])
      pltpu.async_copy(ew_hbm.at[w].at[pl.ds(k * CHUNK, CHUNK)],
                       ewv[b], sew[b])

    def wait_idx(k, b):
      pltpu.make_async_copy(src_hbm.at[w].at[k], srcv[b], ssr[b]).wait()
      pltpu.make_async_copy(dst_hbm.at[w].at[k], dstv[b], sds[b]).wait()
      pltpu.make_async_copy(ew_hbm.at[w].at[pl.ds(k * CHUNK, CHUNK)],
                            ewv[b], sew[b]).wait()

    def issue_gather(b):
      pltpu.async_copy(sup_hbm.at[srcv[b].at[0]], rows[b], sg[b])

    def wait_gather(b):
      pltpu.make_async_copy(sup_hbm.at[srcv[b].at[0]], rows[b], sg[b]).wait()

    def scale_scatter(b):
      @plsc.parallel_loop(0, CHUNK, unroll=4)
      def edge_body(i):
        w16 = plsc.load_gather(ewv[b], [jnp.full((16,), i, jnp.int32)])
        for g in range(D // 32):
          ab = rows[b][i, pl.ds(16 * g, 16)]
          lo = plsc.bitcast(jnp.left_shift(ab, 16), jnp.float32)
          hi = plsc.bitcast(jnp.bitwise_and(ab, jnp.int32(-65536)),
                            jnp.float32)
          frows[i, pl.ds(32 * g, 16)] = lo * w16
          frows[i, pl.ds(32 * g + 16, 16)] = hi * w16
      pltpu.sync_copy(frows, acc.at[dstv[b].at[0]], add=True)

    issue_idx(0, 0)
    wait_idx(0, 0)
    issue_idx(1, 1)
    issue_gather(0)

    def pipe_body(j, _):
      for b in range(2):
        k = 2 * j + b
        nb = 1 - b
        wait_idx(k + 1, nb)
        issue_gather(nb)
        wait_gather(b)
        scale_scatter(b)
        issue_idx(k + 2, b)
      return 0
    lax.fori_loop(0, NCHUNK // 2 - 1, pipe_body, 0)

    wait_idx(NCHUNK - 1, 1)
    issue_gather(1)
    wait_gather(0)
    scale_scatter(0)
    wait_gather(1)
    scale_scatter(1)

    plsc.subcore_barrier()

    def drain(j, _):
      base = (s + j * 16) * DRAIN
      pltpu.sync_copy(acc.at[pl.ds(base, DRAIN)], frows.at[pl.ds(0, DRAIN)])
      pltpu.sync_copy(frows.at[pl.ds(0, DRAIN)],
                      out_hbm.at[c].at[pl.ds(base, DRAIN)])
      return 0
    lax.fori_loop(0, n_rr, drain, 0)

  return spmm


_spmm128 = _make_spmm(NHID)
_spmm64 = _make_spmm(NCLS_PAD)

_RB = 1000


def _pack_bf16(r):
  h = r.shape[1] // 2
  v = lax.bitcast_convert_type(r, jnp.uint32) + jnp.uint32(0x8000)
  packed = (v[:, :h] >> 16) | (v[:, h:] & jnp.uint32(0xFFFF0000))
  return lax.bitcast_convert_type(packed, jnp.int32)


def _mm1_body(x_ref, w_ref, o_ref):
  o_ref[...] = _pack_bf16(jnp.dot(x_ref[...], w_ref[...],
                                  preferred_element_type=jnp.float32))


def _mm1(x, W1):
  return pl.pallas_call(
      _mm1_body,
      grid=(N_NODES // _RB,),
      in_specs=[
          pl.BlockSpec((_RB, NFEAT), lambda i: (i, 0)),
          pl.BlockSpec((NFEAT, NHID), lambda i: (0, 0)),
      ],
      out_specs=pl.BlockSpec((_RB, NHID // 2), lambda i: (i, 0)),
      out_shape=jax.ShapeDtypeStruct((N_NODES, NHID // 2), jnp.int32),
  )(x, W1)


def _combine_body(p0_ref, p1_ref, b_ref, w_ref, o_ref):
  h = jnp.maximum(p0_ref[...] + p1_ref[...] + b_ref[...], 0.0)
  o_ref[...] = _pack_bf16(jnp.dot(h, w_ref[...],
                                  preferred_element_type=jnp.float32))


def _combine(p0, p1, b1, W2p):
  return pl.pallas_call(
      _combine_body,
      grid=(N_NODES // _RB,),
      in_specs=[
          pl.BlockSpec((_RB, NHID), lambda i: (i, 0)),
          pl.BlockSpec((_RB, NHID), lambda i: (i, 0)),
          pl.BlockSpec((1, NHID), lambda i: (0, 0)),
          pl.BlockSpec((NHID, NCLS_PAD), lambda i: (0, 0)),
      ],
      out_specs=pl.BlockSpec((_RB, NCLS_PAD // 2), lambda i: (i, 0)),
      out_shape=jax.ShapeDtypeStruct((N_NODES, NCLS_PAD // 2), jnp.int32),
  )(p0, p1, b1.reshape(1, NHID), W2p)


def _final_body(p0_ref, p1_ref, b_ref, o_ref):
  z = (p0_ref[...] + p1_ref[...])[:, :NCLASS] + b_ref[...]
  z = z - jnp.max(z, axis=1, keepdims=True)
  e = jnp.exp(z)
  o_ref[...] = e / jnp.sum(e, axis=1, keepdims=True)


def _final(p0, p1, b2):
  return pl.pallas_call(
      _final_body,
      grid=(N_NODES // _RB,),
      in_specs=[
          pl.BlockSpec((_RB, NCLS_PAD), lambda i: (i, 0)),
          pl.BlockSpec((_RB, NCLS_PAD), lambda i: (i, 0)),
          pl.BlockSpec((1, NCLASS), lambda i: (0, 0)),
      ],
      out_specs=pl.BlockSpec((_RB, NCLASS), lambda i: (i, 0)),
      out_shape=jax.ShapeDtypeStruct((N_NODES, NCLASS), jnp.float32),
  )(p0, p1, b2.reshape(1, NCLASS))


@jax.jit
def kernel(x, edge_index, edge_weight, W1, b1, W2, b2):
  ei = jnp.pad(edge_index.astype(jnp.int32), ((0, 0), (0, E_PAD - N_EDGES)))
  src = ei[0].reshape(NW, NCHUNK, 1, CHUNK)
  dst = ei[1].reshape(NW, NCHUNK, 1, CHUNK)
  ew = jnp.pad(edge_weight, (0, E_PAD - N_EDGES)).reshape(NW, EDGES_PER_W)

  W1p = W1[:, _INV_T128]
  W2p = jnp.pad(W2, ((0, 0), (0, NCLS_PAD - NCLASS)))[:, _INV_T64]

  sup1 = _mm1(x, W1p)
  p1 = _spmm128(sup1, src, dst, ew)
  sup2 = _combine(p1[0], p1[1], b1, W2p)
  p2 = _spmm64(sup2, src, dst, ew)
  return _final(p2[0], p2[1], b2)
